# baseline (speedup 1.0000x reference)
import functools

import jax
import jax.numpy as jnp
from jax import lax
from jax.experimental import pallas as pl
from jax.experimental.pallas import tpu as pltpu
from jax.experimental.pallas import tpu_sc as plsc

N = 10000
E = 320000
H = 128
D_IN = 128
NPAD = 10240
NSC = 2
NTILES = 16
NW = NSC * NTILES
CH = 80
EPT = E // NW
NCH = EPT // CH

_MESH = plsc.VectorSubcoreMesh(core_axis_name="c", subcore_axis_name="s")


NB = 5
BLK = EPT // NB


def _sc_degree(dst):
    @functools.partial(
        pl.kernel,
        out_type=jax.ShapeDtypeStruct((NSC, NPAD), jnp.float32),
        mesh=_MESH,
        scratch_types=[
            pltpu.VMEM((EPT,), jnp.int32),
            pltpu.VMEM((CH,), jnp.float32),
            pltpu.VMEM((640,), jnp.float32),
            pltpu.VMEM_SHARED((NPAD,), jnp.float32),
            pltpu.SemaphoreType.DMA,
            pltpu.SemaphoreType.DMA,
            pltpu.SemaphoreType.DMA,
        ],
    )
    def k(dst_hbm, out_hbm, dst_all, ones_v, zb_v, acc, sem_i, sem_s0, sem_s1):
        c = lax.axis_index("c")
        s = lax.axis_index("s")
        wid = c * NTILES + s
        base = wid * EPT
        sem_s = (sem_s0, sem_s1)

        for j in range(NB):
            off = pl.multiple_of(base + j * BLK, 8)
            pltpu.async_copy(dst_hbm.at[pl.ds(off, BLK)],
                             dst_all.at[pl.ds(j * BLK, BLK)], sem_i)
        for j in range(CH // 16):
            ones_v[pl.ds(j * 16, 16)] = jnp.full((16,), 1.0, jnp.float32)
        for j in range(640 // 16):
            zb_v[pl.ds(j * 16, 16)] = jnp.zeros((16,), jnp.float32)
        pltpu.sync_copy(zb_v, acc.at[pl.ds(s * 640, 640)])
        for j in range(NB):
            pltpu.make_async_copy(dst_hbm.at[pl.ds(0, BLK)],
                                  dst_all.at[pl.ds(0, BLK)], sem_i).wait()
        plsc.subcore_barrier()

        def didx(r):
            return dst_all.at[pl.ds(pl.multiple_of(r * CH, 8), CH)]

        def start_s(r, b):
            pltpu.async_copy(ones_v, acc.at[didx(r)], sem_s[b], add=True)

        def wait_s(b):
            pltpu.make_async_copy(ones_v, acc.at[didx(0)], sem_s[b]).wait()

        start_s(0, 0)
        start_s(1, 1)

        def body(i, carry):
            r = 2 * i + 2
            wait_s(0)
            start_s(r, 0)
            wait_s(1)
            start_s(r + 1, 1)
            return carry

        lax.fori_loop(0, 61, body, 0)
        wait_s(0)
        start_s(124, 0)
        wait_s(1)
        wait_s(0)
        plsc.subcore_barrier()
        pltpu.sync_copy(acc.at[pl.ds(s * 640, 640)],
                        out_hbm.at[c, pl.ds(s * 640, 640)])

    return k(dst)


def _sc_scatter(zp, src, dst):
    @functools.partial(
        pl.kernel,
        out_type=jax.ShapeDtypeStruct((NSC, NPAD, H), jnp.float32),
        mesh=_MESH,
        scratch_types=(
            [pltpu.VMEM((CH,), jnp.int32) for _ in range(8)]
            + [pltpu.VMEM((CH,), jnp.int32) for _ in range(8)]
            + [pltpu.VMEM((CH, H), jnp.float32) for _ in range(4)]
            + [pltpu.VMEM_SHARED((NPAD, H), jnp.float32)]
            + [pltpu.SemaphoreType.DMA for _ in range(16)]
        ),
    )
    def k(zp_hbm, src_hbm, dst_hbm, out_hbm, *refs):
        srcv = refs[0:8]
        dstv = refs[8:16]
        gbuf = refs[16:20]
        acc = refs[20]
        sem_i = refs[21:29]
        sem_g = refs[29:33]
        sem_s = refs[33:37]
        c = lax.axis_index("c")
        s = lax.axis_index("s")
        wid = c * NTILES + s
        base = wid * EPT

        def start_i(r, m):
            off = pl.multiple_of(base, 8) + jnp.minimum(r, NCH - 1) * CH
            off = pl.multiple_of(off, 8)
            pltpu.async_copy(src_hbm.at[pl.ds(off, CH)], srcv[m], sem_i[m])
            pltpu.async_copy(dst_hbm.at[pl.ds(off, CH)], dstv[m], sem_i[m])

        def wait_i(m):
            pltpu.make_async_copy(src_hbm.at[pl.ds(0, CH)], srcv[m],
                                  sem_i[m]).wait()
            pltpu.make_async_copy(dst_hbm.at[pl.ds(0, CH)], dstv[m],
                                  sem_i[m]).wait()

        def start_g(m, b):
            pltpu.async_copy(zp_hbm.at[srcv[m]], gbuf[b], sem_g[b])

        def wait_g(b):
            pltpu.make_async_copy(zp_hbm.at[srcv[0]], gbuf[b], sem_g[b]).wait()

        def start_s(m, b):
            pltpu.async_copy(gbuf[b], acc.at[dstv[m]], sem_s[b], add=True)

        def wait_s(b):
            pltpu.make_async_copy(gbuf[b], acc.at[dstv[0]], sem_s[b]).wait()

        for m in range(4):
            start_i(m, m)

        def zrow(i, carry):
            for j in range(H // 16):
                gbuf[0][i, pl.ds(j * 16, 16)] = jnp.zeros((16,), jnp.float32)
            return carry

        lax.fori_loop(0, CH, zrow, 0)
        for t in range(640 // CH):
            pltpu.sync_copy(gbuf[0], acc.at[pl.ds(s * 640 + t * CH, CH)])
        plsc.subcore_barrier()

        start_i(4, 4); wait_i(0); start_g(0, 0)
        start_i(5, 5); wait_i(1); start_g(1, 1)
        start_i(6, 6); wait_i(2); start_g(2, 2)
        wait_g(0); start_s(0, 0)
        start_i(7, 7); wait_i(3); start_g(3, 3)
        wait_g(1); start_s(1, 1)

        def body(i, carry):
            q0 = 8 * i + 4
            for u in range(8):
                q = q0 + u
                wait_s(u % 4)
                start_i(q + 4, u)
                wait_i((4 + u) % 8)
                start_g((4 + u) % 8, u % 4)
                wait_g((u + 2) % 4)
                start_s((2 + u) % 8, (u + 2) % 4)
            return carry

        lax.fori_loop(0, 15, body, 0)
        wait_s(0)
        wait_i(4)
        start_g(4, 0)
        wait_g(2); start_s(2, 2)
        wait_g(3); start_s(3, 3)
        wait_g(0); start_s(4, 0)
        wait_s(1); wait_s(2); wait_s(3); wait_s(0)
        wait_i(5); wait_i(6); wait_i(7)
        plsc.subcore_barrier()
        pltpu.sync_copy(acc.at[pl.ds(s * 640, 640)],
                        out_hbm.at[c, pl.ds(s * 640, 640)])

    return k(zp, src, dst)


_R = 10000


def _rspec(ncols):
    return pl.BlockSpec((_R, ncols), lambda i: (i, 0))


def _wspec(shape):
    return pl.BlockSpec(shape, lambda i: (0, 0))


def _pspec(core):
    return pl.BlockSpec((1, _R, H), lambda i, core=core: (core, i, 0))


def _dspec(core):
    return pl.BlockSpec((1, _R, 1), lambda i, core=core: (core, i, 0))


def _dinv(dp0, dp1):
    return lax.rsqrt(dp0[0] + dp1[0] + 1.0)


def _tc_pre(x, W_pre, b_pre, W_c0, degp3):
    def body(x_ref, wp, bp, wc, dp0, dp1, h_ref, z_ref):
        h = jnp.dot(x_ref[...], wp[...], preferred_element_type=jnp.float32) + bp[...]
        h_ref[...] = h
        z_ref[...] = (
            jnp.dot(h, wc[...], preferred_element_type=jnp.float32)
            * _dinv(dp0, dp1)
        )

    return pl.pallas_call(
        body,
        grid=(N // _R,),
        in_specs=[
            _rspec(D_IN), _wspec((D_IN, H)), _wspec((1, H)),
            _wspec((H, H)), _dspec(0), _dspec(1),
        ],
        out_specs=[_rspec(H), _rspec(H)],
        out_shape=[jax.ShapeDtypeStruct((N, H), jnp.float32),
                   jax.ShapeDtypeStruct((N, H), jnp.float32)],
    )(x, W_pre, b_pre[None, :], W_c0, degp3, degp3)


def _tc_partial(hs, Wn):
    nh = len(hs)

    def body(*refs):
        h_refs = refs[:nh]
        w_refs = refs[nh:2 * nh]
        o_ref = refs[-1]
        acc = jnp.dot(h_refs[0][...], w_refs[0][...],
                      preferred_element_type=jnp.float32)
        for j in range(1, nh):
            acc += jnp.dot(h_refs[j][...], w_refs[j][...],
                           preferred_element_type=jnp.float32)
        o_ref[...] = acc

    return pl.pallas_call(
        body,
        grid=(N // _R,),
        in_specs=[_rspec(H)] * nh + [_wspec((H, H))] * nh,
        out_specs=_rspec(H),
        out_shape=jax.ShapeDtypeStruct((N, H), jnp.float32),
    )(*hs, *[Wn[j * H:(j + 1) * H] for j in range(nh)])


def _tc_mid(parts, zp, degp3, b, zpre, Wl):
    def body(p0, p1, z_ref, dp0, dp1, bias_ref, zpre_ref, wl, h_ref, zn_ref):
        d = _dinv(dp0, dp1)
        h = (p0[0] + p1[0] + z_ref[...]) * d + bias_ref[...]
        h = jnp.maximum(h, 0.0)
        h_ref[...] = h
        zn_ref[...] = (
            zpre_ref[...]
            + jnp.dot(h, wl[...], preferred_element_type=jnp.float32)
        ) * d

    return pl.pallas_call(
        body,
        grid=(N // _R,),
        in_specs=[_pspec(0), _pspec(1), _rspec(H), _dspec(0), _dspec(1),
                  _wspec((1, H)), _rspec(H), _wspec((H, H))],
        out_specs=[_rspec(H), _rspec(H)],
        out_shape=[jax.ShapeDtypeStruct((N, H), jnp.float32),
                   jax.ShapeDtypeStruct((N, H), jnp.float32)],
    )(parts, parts, zp, degp3, degp3, b[None, :], zpre, Wl)


def _tc_tail(parts, zp, degp3, b, zpre, Wl, b1, W2, b2, W3, b3, W4, b4):
    def body(p0, p1, z_ref, dp0, dp1, bias_ref, zpre_ref, wl,
             bb1, w2, bb2, w3, bb3, w4, bb4, o_ref):
        d = _dinv(dp0, dp1)
        h = (p0[0] + p1[0] + z_ref[...]) * d + bias_ref[...]
        h = jnp.maximum(h, 0.0)
        t = (zpre_ref[...]
             + jnp.dot(h, wl[...], preferred_element_type=jnp.float32)
             + bb1[...])
        t = jnp.where(t > 0, t, 0.1 * t)
        t = jnp.dot(t, w2[...], preferred_element_type=jnp.float32) + bb2[...]
        t = jnp.maximum(t, 0.0)
        t = jnp.dot(t, w3[...], preferred_element_type=jnp.float32) + bb3[...]
        t = jnp.maximum(t, 0.0)
        o_ref[...] = jnp.dot(t, w4[...], preferred_element_type=jnp.float32) + bb4[...]

    return pl.pallas_call(
        body,
        grid=(N // _R,),
        in_specs=[_pspec(0), _pspec(1), _rspec(H), _dspec(0), _dspec(1),
                  _wspec((1, H)), _rspec(H), _wspec((H, H)),
                  _wspec((1, H)), _wspec((H, H)), _wspec((1, H)),
                  _wspec((H, 256)), _wspec((1, 256)),
                  _wspec((256, H)), _wspec((1, H))],
        out_specs=_rspec(H),
        out_shape=jax.ShapeDtypeStruct((N, H), jnp.float32),
    )(parts, parts, zp, degp3, degp3, b[None, :], zpre, Wl,
      b1[None, :], W2, b2[None, :], W3, b3[None, :], W4, b4[None, :])


def kernel(x, edge_index, W_pre, b_pre, W_c0, b_c0, W_c1, b_c1, W_c2, b_c2,
           W_p1, b_p1, W_p2, b_p2, W_p3, b_p3, W_p4, b_p4):
    src = edge_index[0]
    dst = edge_index[1]

    degp = _sc_degree(dst)
    degp3 = degp.reshape(NSC, NPAD, 1)

    h_pre, zp0 = _tc_pre(x, W_pre, b_pre, W_c0, degp3)
    parts0 = _sc_scatter(zp0, src, dst)
    zpre1 = _tc_partial([h_pre], W_c1)
    h0, zp1 = _tc_mid(parts0, zp0, degp3, b_c0, zpre1, W_c1[H:])
    parts1 = _sc_scatter(zp1, src, dst)
    zpre2 = _tc_partial([h_pre, h0], W_c2)
    h1, zp2 = _tc_mid(parts1, zp1, degp3, b_c1, zpre2, W_c2[2 * H:])
    parts2 = _sc_scatter(zp2, src, dst)
    zpreH = _tc_partial([h_pre, h0, h1], W_p1)
    return _tc_tail(parts2, zp2, degp3, b_c2, zpreH, W_p1[3 * H:],
                    b_p1, W_p2, b_p2, W_p3, b_p3, W_p4, b_p4)


# speedup vs baseline: 1.0184x; 1.0184x over previous
import functools

import jax
import jax.numpy as jnp
from jax import lax
from jax.experimental import pallas as pl
from jax.experimental.pallas import tpu as pltpu
from jax.experimental.pallas import tpu_sc as plsc

N = 10000
E = 320000
H = 128
D_IN = 128
NPAD = 10240
NSC = 2
NTILES = 16
NW = NSC * NTILES
CH = 80
EPT = E // NW
NCH = EPT // CH

_MESH = plsc.VectorSubcoreMesh(core_axis_name="c", subcore_axis_name="s")


NB = 5
BLK = EPT // NB


def _sc_degree(dst):
    @functools.partial(
        pl.kernel,
        out_type=jax.ShapeDtypeStruct((NSC, NPAD), jnp.float32),
        mesh=_MESH,
        scratch_types=[
            pltpu.VMEM((EPT,), jnp.int32),
            pltpu.VMEM((CH,), jnp.float32),
            pltpu.VMEM((640,), jnp.float32),
            pltpu.VMEM_SHARED((NPAD,), jnp.float32),
            pltpu.SemaphoreType.DMA,
            pltpu.SemaphoreType.DMA,
            pltpu.SemaphoreType.DMA,
        ],
    )
    def k(dst_hbm, out_hbm, dst_all, ones_v, zb_v, acc, sem_i, sem_s0, sem_s1):
        c = lax.axis_index("c")
        s = lax.axis_index("s")
        wid = c * NTILES + s
        base = wid * EPT
        sem_s = (sem_s0, sem_s1)

        for j in range(NB):
            off = pl.multiple_of(base + j * BLK, 8)
            pltpu.async_copy(dst_hbm.at[pl.ds(off, BLK)],
                             dst_all.at[pl.ds(j * BLK, BLK)], sem_i)
        for j in range(CH // 16):
            ones_v[pl.ds(j * 16, 16)] = jnp.full((16,), 1.0, jnp.float32)
        for j in range(640 // 16):
            zb_v[pl.ds(j * 16, 16)] = jnp.zeros((16,), jnp.float32)
        pltpu.sync_copy(zb_v, acc.at[pl.ds(s * 640, 640)])
        for j in range(NB):
            pltpu.make_async_copy(dst_hbm.at[pl.ds(0, BLK)],
                                  dst_all.at[pl.ds(0, BLK)], sem_i).wait()
        plsc.subcore_barrier()

        def didx(r):
            return dst_all.at[pl.ds(pl.multiple_of(r * CH, 8), CH)]

        def start_s(r, b):
            pltpu.async_copy(ones_v, acc.at[didx(r)], sem_s[b], add=True)

        def wait_s(b):
            pltpu.make_async_copy(ones_v, acc.at[didx(0)], sem_s[b]).wait()

        start_s(0, 0)
        start_s(1, 1)

        def body(i, carry):
            r = 2 * i + 2
            wait_s(0)
            start_s(r, 0)
            wait_s(1)
            start_s(r + 1, 1)
            return carry

        lax.fori_loop(0, 61, body, 0)
        wait_s(0)
        start_s(124, 0)
        wait_s(1)
        wait_s(0)
        plsc.subcore_barrier()
        pltpu.sync_copy(acc.at[pl.ds(s * 640, 640)],
                        out_hbm.at[c, pl.ds(s * 640, 640)])

    return k(dst)


def _sc_scatter(zp, src, dst):
    @functools.partial(
        pl.kernel,
        out_type=jax.ShapeDtypeStruct((NSC, NPAD, H), jnp.float32),
        mesh=_MESH,
        scratch_types=(
            [pltpu.VMEM((CH,), jnp.int32) for _ in range(8)]
            + [pltpu.VMEM((CH,), jnp.int32) for _ in range(8)]
            + [pltpu.VMEM((CH, H), jnp.float32) for _ in range(4)]
            + [pltpu.VMEM_SHARED((NPAD, H), jnp.float32)]
            + [pltpu.SemaphoreType.DMA for _ in range(16)]
        ),
    )
    def k(zp_hbm, src_hbm, dst_hbm, out_hbm, *refs):
        srcv = refs[0:8]
        dstv = refs[8:16]
        gbuf = refs[16:20]
        acc = refs[20]
        sem_i = refs[21:29]
        sem_g = refs[29:33]
        sem_s = refs[33:37]
        c = lax.axis_index("c")
        s = lax.axis_index("s")
        wid = c * NTILES + s
        base = wid * EPT

        def start_i(r, m):
            off = pl.multiple_of(base, 8) + jnp.minimum(r, NCH - 1) * CH
            off = pl.multiple_of(off, 8)
            pltpu.async_copy(src_hbm.at[pl.ds(off, CH)], srcv[m], sem_i[m])
            pltpu.async_copy(dst_hbm.at[pl.ds(off, CH)], dstv[m], sem_i[m])

        def wait_i(m):
            pltpu.make_async_copy(src_hbm.at[pl.ds(0, CH)], srcv[m],
                                  sem_i[m]).wait()
            pltpu.make_async_copy(dst_hbm.at[pl.ds(0, CH)], dstv[m],
                                  sem_i[m]).wait()

        def start_g(m, b):
            pltpu.async_copy(zp_hbm.at[srcv[m]], gbuf[b], sem_g[b])

        def wait_g(b):
            pltpu.make_async_copy(zp_hbm.at[srcv[0]], gbuf[b], sem_g[b]).wait()

        def start_s(m, b):
            pltpu.async_copy(gbuf[b], acc.at[dstv[m]], sem_s[b], add=True)

        def wait_s(b):
            pltpu.make_async_copy(gbuf[b], acc.at[dstv[0]], sem_s[b]).wait()

        for m in range(4):
            start_i(m, m)

        def zrow(i, carry):
            for j in range(H // 16):
                gbuf[0][i, pl.ds(j * 16, 16)] = jnp.zeros((16,), jnp.float32)
            return carry

        lax.fori_loop(0, CH, zrow, 0)
        for t in range(640 // CH):
            pltpu.sync_copy(gbuf[0], acc.at[pl.ds(s * 640 + t * CH, CH)])
        plsc.subcore_barrier()

        start_i(4, 4); wait_i(0); start_g(0, 0)
        start_i(5, 5); wait_i(1); start_g(1, 1)
        start_i(6, 6); wait_i(2); start_g(2, 2)
        wait_g(0); start_s(0, 0)
        start_i(7, 7); wait_i(3); start_g(3, 3)
        wait_g(1); start_s(1, 1)

        def body(i, carry):
            q0 = 8 * i + 4
            for u in range(8):
                q = q0 + u
                wait_s(u % 4)
                start_i(q + 4, u)
                wait_i((4 + u) % 8)
                start_g((4 + u) % 8, u % 4)
                wait_g((u + 2) % 4)
                start_s((2 + u) % 8, (u + 2) % 4)
            return carry

        lax.fori_loop(0, 15, body, 0)
        wait_s(0)
        wait_i(4)
        start_g(4, 0)
        wait_g(2); start_s(2, 2)
        wait_g(3); start_s(3, 3)
        wait_g(0); start_s(4, 0)
        wait_s(1); wait_s(2); wait_s(3); wait_s(0)
        wait_i(5); wait_i(6); wait_i(7)
        plsc.subcore_barrier()
        pltpu.sync_copy(acc.at[pl.ds(s * 640, 640)],
                        out_hbm.at[c, pl.ds(s * 640, 640)])

    return k(zp, src, dst)


_R = 5000


def _rspec(ncols):
    return pl.BlockSpec((_R, ncols), lambda i: (i, 0))


def _wspec(shape):
    return pl.BlockSpec(shape, lambda i: (0, 0))


def _pspec(core):
    return pl.BlockSpec((1, _R, H), lambda i, core=core: (core, i, 0))


def _dspec(core):
    return pl.BlockSpec((1, _R, 1), lambda i, core=core: (core, i, 0))


def _dinv(dp0, dp1):
    return lax.rsqrt(dp0[0] + dp1[0] + 1.0)


def _tc_linpre(x, W_pre, b_pre):
    def body(x_ref, wp, bp, h_ref):
        h_ref[...] = (
            jnp.dot(x_ref[...], wp[...], preferred_element_type=jnp.float32)
            + bp[...]
        )

    return pl.pallas_call(
        body,
        grid=(N // _R,),
        in_specs=[_rspec(D_IN), _wspec((D_IN, H)), _wspec((1, H))],
        out_specs=_rspec(H),
        out_shape=jax.ShapeDtypeStruct((N, H), jnp.float32),
    )(x, W_pre, b_pre[None, :])


def _tc_scale0(h_pre, W_c0, degp3):
    def body(h_ref, wc, dp0, dp1, z_ref):
        z_ref[...] = (
            jnp.dot(h_ref[...], wc[...], preferred_element_type=jnp.float32)
            * _dinv(dp0, dp1)
        )

    return pl.pallas_call(
        body,
        grid=(N // _R,),
        in_specs=[_rspec(H), _wspec((H, H)), _dspec(0), _dspec(1)],
        out_specs=_rspec(H),
        out_shape=jax.ShapeDtypeStruct((N, H), jnp.float32),
    )(h_pre, W_c0, degp3, degp3)


def _tc_partial(hs, Wn):
    nh = len(hs)

    def body(*refs):
        h_refs = refs[:nh]
        w_refs = refs[nh:2 * nh]
        o_ref = refs[-1]
        acc = jnp.dot(h_refs[0][...], w_refs[0][...],
                      preferred_element_type=jnp.float32)
        for j in range(1, nh):
            acc += jnp.dot(h_refs[j][...], w_refs[j][...],
                           preferred_element_type=jnp.float32)
        o_ref[...] = acc

    return pl.pallas_call(
        body,
        grid=(N // _R,),
        in_specs=[_rspec(H)] * nh + [_wspec((H, H))] * nh,
        out_specs=_rspec(H),
        out_shape=jax.ShapeDtypeStruct((N, H), jnp.float32),
    )(*hs, *[Wn[j * H:(j + 1) * H] for j in range(nh)])


def _tc_mid(parts, zp, degp3, b, zpre, Wl):
    def body(p0, p1, z_ref, dp0, dp1, bias_ref, zpre_ref, wl, h_ref, zn_ref):
        d = _dinv(dp0, dp1)
        h = (p0[0] + p1[0] + z_ref[...]) * d + bias_ref[...]
        h = jnp.maximum(h, 0.0)
        h_ref[...] = h
        zn_ref[...] = (
            zpre_ref[...]
            + jnp.dot(h, wl[...], preferred_element_type=jnp.float32)
        ) * d

    return pl.pallas_call(
        body,
        grid=(N // _R,),
        in_specs=[_pspec(0), _pspec(1), _rspec(H), _dspec(0), _dspec(1),
                  _wspec((1, H)), _rspec(H), _wspec((H, H))],
        out_specs=[_rspec(H), _rspec(H)],
        out_shape=[jax.ShapeDtypeStruct((N, H), jnp.float32),
                   jax.ShapeDtypeStruct((N, H), jnp.float32)],
    )(parts, parts, zp, degp3, degp3, b[None, :], zpre, Wl)


def _tc_tail(parts, zp, degp3, b, zpre, Wl, b1, W2, b2, W3, b3, W4, b4):
    def body(p0, p1, z_ref, dp0, dp1, bias_ref, zpre_ref, wl,
             bb1, w2, bb2, w3, bb3, w4, bb4, o_ref):
        d = _dinv(dp0, dp1)
        h = (p0[0] + p1[0] + z_ref[...]) * d + bias_ref[...]
        h = jnp.maximum(h, 0.0)
        t = (zpre_ref[...]
             + jnp.dot(h, wl[...], preferred_element_type=jnp.float32)
             + bb1[...])
        t = jnp.where(t > 0, t, 0.1 * t)
        t = jnp.dot(t, w2[...], preferred_element_type=jnp.float32) + bb2[...]
        t = jnp.maximum(t, 0.0)
        t = jnp.dot(t, w3[...], preferred_element_type=jnp.float32) + bb3[...]
        t = jnp.maximum(t, 0.0)
        o_ref[...] = jnp.dot(t, w4[...], preferred_element_type=jnp.float32) + bb4[...]

    return pl.pallas_call(
        body,
        grid=(N // _R,),
        in_specs=[_pspec(0), _pspec(1), _rspec(H), _dspec(0), _dspec(1),
                  _wspec((1, H)), _rspec(H), _wspec((H, H)),
                  _wspec((1, H)), _wspec((H, H)), _wspec((1, H)),
                  _wspec((H, 256)), _wspec((1, 256)),
                  _wspec((256, H)), _wspec((1, H))],
        out_specs=_rspec(H),
        out_shape=jax.ShapeDtypeStruct((N, H), jnp.float32),
    )(parts, parts, zp, degp3, degp3, b[None, :], zpre, Wl,
      b1[None, :], W2, b2[None, :], W3, b3[None, :], W4, b4[None, :])


def kernel(x, edge_index, W_pre, b_pre, W_c0, b_c0, W_c1, b_c1, W_c2, b_c2,
           W_p1, b_p1, W_p2, b_p2, W_p3, b_p3, W_p4, b_p4):
    src = edge_index[0]
    dst = edge_index[1]

    degp = _sc_degree(dst)
    h_pre = _tc_linpre(x, W_pre, b_pre)
    degp3 = degp.reshape(NSC, NPAD, 1)

    zp0 = _tc_scale0(h_pre, W_c0, degp3)
    parts0 = _sc_scatter(zp0, src, dst)
    zpre1 = _tc_partial([h_pre], W_c1)
    h0, zp1 = _tc_mid(parts0, zp0, degp3, b_c0, zpre1, W_c1[H:])
    parts1 = _sc_scatter(zp1, src, dst)
    zpre2 = _tc_partial([h_pre, h0], W_c2)
    h1, zp2 = _tc_mid(parts1, zp1, degp3, b_c1, zpre2, W_c2[2 * H:])
    parts2 = _sc_scatter(zp2, src, dst)
    zpreH = _tc_partial([h_pre, h0, h1], W_p1)
    return _tc_tail(parts2, zp2, degp3, b_c2, zpreH, W_p1[3 * H:],
                    b_p1, W_p2, b_p2, W_p3, b_p3, W_p4, b_p4)
